# baseline (speedup 1.0000x reference)
import functools

import jax
import jax.numpy as jnp
from jax import lax
from jax.experimental import pallas as pl
from jax.experimental.pallas import tpu as pltpu
from jax.experimental.pallas import tpu_sc as plsc


def _sc_gather_rows(table, idx):
    T = idx.shape[0]
    D = table.shape[1]
    info = plsc.get_sparse_core_info()
    nc, ns = info.num_cores, info.num_subcores
    nw = nc * ns
    rows_per_w = T // nw

    mesh = plsc.VectorSubcoreMesh(core_axis_name="c", subcore_axis_name="s")

    @functools.partial(
        pl.kernel,
        mesh=mesh,
        out_type=jax.ShapeDtypeStruct((T, D), jnp.float32),
        scratch_types=[
            pltpu.VMEM((rows_per_w,), jnp.int32),
            pltpu.VMEM((rows_per_w, D), jnp.float32),
            pltpu.SemaphoreType.DMA,
        ],
    )
    def gather(table_hbm, idx_hbm, out_hbm, idx_v, rows_v, sem):
        wid = lax.axis_index("s") * nc + lax.axis_index("c")
        base = wid * rows_per_w
        pltpu.sync_copy(idx_hbm.at[pl.ds(base, rows_per_w)], idx_v)
        pltpu.async_copy(table_hbm.at[idx_v], rows_v, sem).wait()
        pltpu.sync_copy(rows_v, out_hbm.at[pl.ds(base, rows_per_w)])

    return gather(table, idx)


def _mlp_body(x_ref, w1_ref, b1_ref, w2_ref, b2_ref, out_ref, h_ref):
    @pl.when(pl.program_id(0) == 0)
    def _compute_h():
        h = jnp.dot(x_ref[...], w1_ref[...], preferred_element_type=jnp.float32)
        h_ref[...] = jnp.tanh(h + b1_ref[...])

    out_ref[...] = (
        jnp.dot(h_ref[...], w2_ref[...], preferred_element_type=jnp.float32)
        + b2_ref[...]
    )


def kernel(prefix, emb, W1, b1, W2, b2):
    B, L = prefix.shape
    V, D = emb.shape
    H = W1.shape[1]
    N = W2.shape[1]
    T = B * L

    TN = 4096
    grid = (N // TN,)

    idx = prefix.reshape(T).astype(jnp.int32)
    x = _sc_gather_rows(emb, idx)

    out = pl.pallas_call(
        _mlp_body,
        grid=grid,
        in_specs=[
            pl.BlockSpec((T, D), lambda j: (0, 0)),
            pl.BlockSpec((D, H), lambda j: (0, 0)),
            pl.BlockSpec((1, H), lambda j: (0, 0)),
            pl.BlockSpec((H, TN), lambda j: (0, j)),
            pl.BlockSpec((1, TN), lambda j: (0, j)),
        ],
        out_specs=pl.BlockSpec((T, TN), lambda j: (0, j)),
        out_shape=jax.ShapeDtypeStruct((T, N), jnp.float32),
        scratch_shapes=[pltpu.VMEM((T, H), jnp.float32)],
    )(x, W1, b1.reshape(1, H), W2, b2.reshape(1, N))

    return out.reshape(B, L, N)


# speedup vs baseline: 1.3288x; 1.3288x over previous
import jax
import jax.numpy as jnp
from jax.experimental import pallas as pl
from jax.experimental.pallas import tpu as pltpu


def _mlp_body(idx_ref, emb_ref, w1_ref, b1_ref, w2_ref, b2_ref, out_ref,
              htab_ref, oh_ref):
    T, V = idx_ref.shape[0], emb_ref.shape[0]

    @pl.when(pl.program_id(0) == 0)
    def _compute_h():
        h = jnp.dot(emb_ref[...], w1_ref[...], preferred_element_type=jnp.float32)
        htab_ref[...] = jnp.tanh(h + b1_ref[...])
        iota = jax.lax.broadcasted_iota(jnp.int32, (T, V), 1)
        oh_ref[...] = jnp.where(iota == idx_ref[...], 1.0, 0.0).astype(jnp.float32)

    m = jnp.dot(htab_ref[...], w2_ref[...], preferred_element_type=jnp.float32)
    out_ref[...] = (
        jnp.dot(oh_ref[...], m, preferred_element_type=jnp.float32) + b2_ref[...]
    )


def kernel(prefix, emb, W1, b1, W2, b2):
    B, L = prefix.shape
    V, D = emb.shape
    H = W1.shape[1]
    N = W2.shape[1]
    T = B * L

    TN = 8192
    grid = (N // TN,)

    idx = prefix.reshape(T, 1).astype(jnp.int32)

    out = pl.pallas_call(
        _mlp_body,
        grid=grid,
        in_specs=[
            pl.BlockSpec((T, 1), lambda j: (0, 0)),
            pl.BlockSpec((V, D), lambda j: (0, 0)),
            pl.BlockSpec((D, H), lambda j: (0, 0)),
            pl.BlockSpec((1, H), lambda j: (0, 0)),
            pl.BlockSpec((H, TN), lambda j: (0, j)),
            pl.BlockSpec((1, TN), lambda j: (0, j)),
        ],
        out_specs=pl.BlockSpec((T, TN), lambda j: (0, j)),
        out_shape=jax.ShapeDtypeStruct((T, N), jnp.float32),
        scratch_shapes=[
            pltpu.VMEM((V, H), jnp.float32),
            pltpu.VMEM((T, V), jnp.float32),
        ],
    )(idx, emb, W1, b1.reshape(1, H), W2, b2.reshape(1, N))

    return out.reshape(B, L, N)


# speedup vs baseline: 1.4390x; 1.0829x over previous
import jax
import jax.numpy as jnp
from jax.experimental import pallas as pl
from jax.experimental.pallas import tpu as pltpu

_TN = 2048
_NBUF = 4


def _mlp_body(idx_ref, emb_ref, w1_ref, b1_ref, b2_ref, w2_hbm, out_hbm,
              htab_ref, oh_ref, w2buf, outbuf, rsem, wsem):
    T, V = idx_ref.shape[0], emb_ref.shape[0]
    N = w2_hbm.shape[1]
    steps = N // _TN

    def read_start(j, slot):
        pltpu.make_async_copy(
            w2_hbm.at[:, pl.ds(j * _TN, _TN)], w2buf.at[slot], rsem.at[slot]
        ).start()

    for p in range(_NBUF):
        read_start(p, p)

    h = jnp.dot(emb_ref[...], w1_ref[...], preferred_element_type=jnp.float32)
    htab_ref[...] = jnp.tanh(h + b1_ref[...])
    iota = jax.lax.broadcasted_iota(jnp.int32, (T, V), 1)
    oh_ref[...] = jnp.where(iota == idx_ref[...], 1.0, 0.0).astype(jnp.float32)

    for j in range(steps):
        slot = j % _NBUF
        pltpu.make_async_copy(
            w2_hbm.at[:, pl.ds(j * _TN, _TN)], w2buf.at[slot], rsem.at[slot]
        ).wait()
        m = jnp.dot(htab_ref[...], w2buf[slot], preferred_element_type=jnp.float32)
        o = (
            jnp.dot(oh_ref[...], m, preferred_element_type=jnp.float32)
            + b2_ref[:, j * _TN:(j + 1) * _TN]
        )
        if j >= _NBUF:
            pltpu.make_async_copy(
                outbuf.at[slot],
                out_hbm.at[:, pl.ds((j - _NBUF) * _TN, _TN)],
                wsem.at[slot],
            ).wait()
        outbuf[slot] = o
        pltpu.make_async_copy(
            outbuf.at[slot], out_hbm.at[:, pl.ds(j * _TN, _TN)], wsem.at[slot]
        ).start()
        if j + _NBUF < steps:
            read_start(j + _NBUF, slot)

    for j in range(max(0, steps - _NBUF), steps):
        slot = j % _NBUF
        pltpu.make_async_copy(
            outbuf.at[slot], out_hbm.at[:, pl.ds(j * _TN, _TN)], wsem.at[slot]
        ).wait()


def kernel(prefix, emb, W1, b1, W2, b2):
    B, L = prefix.shape
    V, D = emb.shape
    H = W1.shape[1]
    N = W2.shape[1]
    T = B * L

    idx = prefix.reshape(T, 1).astype(jnp.int32)

    out = pl.pallas_call(
        _mlp_body,
        in_specs=[
            pl.BlockSpec(memory_space=pltpu.MemorySpace.VMEM),
            pl.BlockSpec(memory_space=pltpu.MemorySpace.VMEM),
            pl.BlockSpec(memory_space=pltpu.MemorySpace.VMEM),
            pl.BlockSpec(memory_space=pltpu.MemorySpace.VMEM),
            pl.BlockSpec(memory_space=pltpu.MemorySpace.VMEM),
            pl.BlockSpec(memory_space=pl.ANY),
        ],
        out_specs=pl.BlockSpec(memory_space=pl.ANY),
        out_shape=jax.ShapeDtypeStruct((T, N), jnp.float32),
        scratch_shapes=[
            pltpu.VMEM((V, H), jnp.float32),
            pltpu.VMEM((T, V), jnp.float32),
            pltpu.VMEM((_NBUF, H, _TN), jnp.float32),
            pltpu.VMEM((_NBUF, T, _TN), jnp.float32),
            pltpu.SemaphoreType.DMA((_NBUF,)),
            pltpu.SemaphoreType.DMA((_NBUF,)),
        ],
    )(idx, emb, W1, b1.reshape(1, H), b2.reshape(1, N), W2)

    return out.reshape(B, L, N)


# speedup vs baseline: 1.4446x; 1.0039x over previous
import jax
import jax.numpy as jnp
from jax.experimental import pallas as pl
from jax.experimental.pallas import tpu as pltpu

_TN = 1024
_NBUF = 6


def _mlp_body(idx_ref, emb_ref, w1_ref, b1_ref, b2_ref, w2_hbm, out_hbm,
              htab_ref, oh_ref, w2buf, outbuf, rsem, wsem):
    T, V = idx_ref.shape[0], emb_ref.shape[0]
    N = w2_hbm.shape[1]
    steps = N // _TN

    def read_start(j, slot):
        pltpu.make_async_copy(
            w2_hbm.at[:, pl.ds(j * _TN, _TN)], w2buf.at[slot], rsem.at[slot]
        ).start()

    for p in range(_NBUF):
        read_start(p, p)

    h = jnp.dot(emb_ref[...], w1_ref[...], preferred_element_type=jnp.float32)
    htab_ref[...] = jnp.tanh(h + b1_ref[...])
    iota = jax.lax.broadcasted_iota(jnp.int32, (T, V), 1)
    oh_ref[...] = jnp.where(iota == idx_ref[...], 1.0, 0.0).astype(jnp.float32)

    for j in range(steps):
        slot = j % _NBUF
        pltpu.make_async_copy(
            w2_hbm.at[:, pl.ds(j * _TN, _TN)], w2buf.at[slot], rsem.at[slot]
        ).wait()
        m = jnp.dot(htab_ref[...], w2buf[slot], preferred_element_type=jnp.float32)
        o = (
            jnp.dot(oh_ref[...], m, preferred_element_type=jnp.float32)
            + b2_ref[:, j * _TN:(j + 1) * _TN]
        )
        if j >= _NBUF:
            pltpu.make_async_copy(
                outbuf.at[slot],
                out_hbm.at[:, pl.ds((j - _NBUF) * _TN, _TN)],
                wsem.at[slot],
            ).wait()
        outbuf[slot] = o
        pltpu.make_async_copy(
            outbuf.at[slot], out_hbm.at[:, pl.ds(j * _TN, _TN)], wsem.at[slot]
        ).start()
        if j + _NBUF < steps:
            read_start(j + _NBUF, slot)

    for j in range(max(0, steps - _NBUF), steps):
        slot = j % _NBUF
        pltpu.make_async_copy(
            outbuf.at[slot], out_hbm.at[:, pl.ds(j * _TN, _TN)], wsem.at[slot]
        ).wait()


def kernel(prefix, emb, W1, b1, W2, b2):
    B, L = prefix.shape
    V, D = emb.shape
    H = W1.shape[1]
    N = W2.shape[1]
    T = B * L

    idx = prefix.reshape(T, 1).astype(jnp.int32)

    out = pl.pallas_call(
        _mlp_body,
        in_specs=[
            pl.BlockSpec(memory_space=pltpu.MemorySpace.VMEM),
            pl.BlockSpec(memory_space=pltpu.MemorySpace.VMEM),
            pl.BlockSpec(memory_space=pltpu.MemorySpace.VMEM),
            pl.BlockSpec(memory_space=pltpu.MemorySpace.VMEM),
            pl.BlockSpec(memory_space=pltpu.MemorySpace.VMEM),
            pl.BlockSpec(memory_space=pl.ANY),
        ],
        out_specs=pl.BlockSpec(memory_space=pl.ANY),
        out_shape=jax.ShapeDtypeStruct((T, N), jnp.float32),
        scratch_shapes=[
            pltpu.VMEM((V, H), jnp.float32),
            pltpu.VMEM((T, V), jnp.float32),
            pltpu.VMEM((_NBUF, H, _TN), jnp.float32),
            pltpu.VMEM((_NBUF, T, _TN), jnp.float32),
            pltpu.SemaphoreType.DMA((_NBUF,)),
            pltpu.SemaphoreType.DMA((_NBUF,)),
        ],
    )(idx, emb, W1, b1.reshape(1, H), b2.reshape(1, N), W2)

    return out.reshape(B, L, N)


# speedup vs baseline: 1.4461x; 1.0011x over previous
import jax
import jax.numpy as jnp
from jax.experimental import pallas as pl
from jax.experimental.pallas import tpu as pltpu

_TN = 512
_NBUF = 8


def _mlp_body(idx_ref, emb_ref, w1_ref, b1_ref, b2_ref, w2_hbm, out_hbm,
              htab_ref, oh_ref, w2buf, outbuf, rsem, wsem):
    T, V = idx_ref.shape[0], emb_ref.shape[0]
    N = w2_hbm.shape[1]
    steps = N // _TN

    def read_start(j, slot):
        pltpu.make_async_copy(
            w2_hbm.at[:, pl.ds(j * _TN, _TN)], w2buf.at[slot], rsem.at[slot]
        ).start()

    for p in range(_NBUF):
        read_start(p, p)

    h = jnp.dot(emb_ref[...], w1_ref[...], preferred_element_type=jnp.float32)
    htab_ref[...] = jnp.tanh(h + b1_ref[...])
    iota = jax.lax.broadcasted_iota(jnp.int32, (T, V), 1)
    oh_ref[...] = jnp.where(iota == idx_ref[...], 1.0, 0.0).astype(jnp.float32)

    for j in range(steps):
        slot = j % _NBUF
        pltpu.make_async_copy(
            w2_hbm.at[:, pl.ds(j * _TN, _TN)], w2buf.at[slot], rsem.at[slot]
        ).wait()
        m = jnp.dot(htab_ref[...], w2buf[slot], preferred_element_type=jnp.float32)
        o = (
            jnp.dot(oh_ref[...], m, preferred_element_type=jnp.float32)
            + b2_ref[:, j * _TN:(j + 1) * _TN]
        )
        if j >= _NBUF:
            pltpu.make_async_copy(
                outbuf.at[slot],
                out_hbm.at[:, pl.ds((j - _NBUF) * _TN, _TN)],
                wsem.at[slot],
            ).wait()
        outbuf[slot] = o
        pltpu.make_async_copy(
            outbuf.at[slot], out_hbm.at[:, pl.ds(j * _TN, _TN)], wsem.at[slot]
        ).start()
        if j + _NBUF < steps:
            read_start(j + _NBUF, slot)

    for j in range(max(0, steps - _NBUF), steps):
        slot = j % _NBUF
        pltpu.make_async_copy(
            outbuf.at[slot], out_hbm.at[:, pl.ds(j * _TN, _TN)], wsem.at[slot]
        ).wait()


def kernel(prefix, emb, W1, b1, W2, b2):
    B, L = prefix.shape
    V, D = emb.shape
    H = W1.shape[1]
    N = W2.shape[1]
    T = B * L

    idx = prefix.reshape(T, 1).astype(jnp.int32)

    out = pl.pallas_call(
        _mlp_body,
        in_specs=[
            pl.BlockSpec(memory_space=pltpu.MemorySpace.VMEM),
            pl.BlockSpec(memory_space=pltpu.MemorySpace.VMEM),
            pl.BlockSpec(memory_space=pltpu.MemorySpace.VMEM),
            pl.BlockSpec(memory_space=pltpu.MemorySpace.VMEM),
            pl.BlockSpec(memory_space=pltpu.MemorySpace.VMEM),
            pl.BlockSpec(memory_space=pl.ANY),
        ],
        out_specs=pl.BlockSpec(memory_space=pl.ANY),
        out_shape=jax.ShapeDtypeStruct((T, N), jnp.float32),
        scratch_shapes=[
            pltpu.VMEM((V, H), jnp.float32),
            pltpu.VMEM((T, V), jnp.float32),
            pltpu.VMEM((_NBUF, H, _TN), jnp.float32),
            pltpu.VMEM((_NBUF, T, _TN), jnp.float32),
            pltpu.SemaphoreType.DMA((_NBUF,)),
            pltpu.SemaphoreType.DMA((_NBUF,)),
        ],
    )(idx, emb, W1, b1.reshape(1, H), b2.reshape(1, N), W2)

    return out.reshape(B, L, N)
